# Initial kernel scaffold; baseline (speedup 1.0000x reference)
import jax
import jax.numpy as jnp
from jax.experimental import pallas as pl


def kernel(x, edge_index, edge_attr, params):
    raise NotImplementedError("write your pallas kernel here")


# speedup vs baseline: 1.3189x; 1.3189x over previous
import functools
import math

import jax
import jax.numpy as jnp
from jax import lax
from jax.experimental import pallas as pl
from jax.experimental.pallas import tpu as pltpu
from jax.experimental.pallas import tpu_sc as plsc

N = 10000
E = 320000
NC = 2
NS = 16
NW = NC * NS
EPW = E // NW
NROW = N // NS
ZR = 125

_MESH = plsc.VectorSubcoreMesh(
    core_axis_name="c", subcore_axis_name="s", num_cores=NC, num_subcores=NS)

_SELU_L = 1.0507009873554804934193349852946
_SELU_A = 1.6732632423543772848170429916717


def _hsum16(v):
    t = [v[i] for i in range(16)]
    while len(t) > 1:
        t = [t[i] + t[i + 1] for i in range(0, len(t), 2)]
    return t[0]


def _splat(ref1d, i):
    v = ref1d[pl.ds(i, 16)]
    return jnp.full((16,), v[0], jnp.float32)


def _zero_fill(buf, rows, groups):
    def body(r, carry):
        for g in range(groups):
            buf[r, pl.ds(g * 16, 16)] = jnp.zeros((16,), jnp.float32)
        return carry
    lax.fori_loop(0, rows, body, 0)


@functools.lru_cache(None)
def _gine_agg(D, K=80):
    G = D // 16
    ITERS = EPW // K

    @functools.partial(
        pl.kernel, mesh=_MESH,
        out_type=jax.ShapeDtypeStruct((NC, N, D), jnp.float32),
        scratch_types=[
            pltpu.VMEM((K,), jnp.int32),
            pltpu.VMEM((K,), jnp.int32),
            pltpu.VMEM((K * 4 + 16,), jnp.float32),
            pltpu.VMEM((K, D), jnp.float32),
            pltpu.VMEM((K, D), jnp.float32),
            pltpu.VMEM((4, D), jnp.float32),
            pltpu.VMEM((1, D), jnp.float32),
            pltpu.VMEM((16, D), jnp.float32),
            pltpu.VMEM_SHARED((N, D), jnp.float32),
            pltpu.SemaphoreType.DMA,
        ])
    def k(h_hbm, src_hbm, dst_hbm, attr_hbm, w_hbm, b_hbm, out_hbm,
          srcv, dstv, attrv, grows, msgs, wv, bv, zbuf, acc, sem):
        core = lax.axis_index("c")
        sid = lax.axis_index("s")
        wid = sid * NC + core
        pltpu.sync_copy(w_hbm, wv)
        pltpu.sync_copy(b_hbm, bv)
        _zero_fill(zbuf, 16, G)
        row0 = sid * 624
        for j in range(40):
            pltpu.sync_copy(zbuf, acc.at[pl.ds(row0 + j * 16, 16)])
        plsc.subcore_barrier()

        base0 = wid * EPW

        def chunk(it, carry):
            base = base0 + it * K
            pltpu.sync_copy(src_hbm.at[pl.ds(base, K)], srcv)
            pltpu.sync_copy(dst_hbm.at[pl.ds(base, K)], dstv)
            pltpu.sync_copy(attr_hbm.at[pl.ds(base * 4, K * 4)], attrv.at[pl.ds(0, K * 4)])
            pltpu.async_copy(h_hbm.at[srcv], grows, sem).wait()

            def edge(e, c2):
                a0 = _splat(attrv, e * 4)
                a1 = _splat(attrv, e * 4 + 1)
                a2 = _splat(attrv, e * 4 + 2)
                a3 = _splat(attrv, e * 4 + 3)
                for g in range(G):
                    s = pl.ds(g * 16, 16)
                    m = grows[e, s] + bv[0, s]
                    m = m + a0 * wv[0, s] + a1 * wv[1, s]
                    m = m + a2 * wv[2, s] + a3 * wv[3, s]
                    msgs[e, s] = jnp.maximum(m, 0.0)
                return c2
            lax.fori_loop(0, K, edge, 0)
            pltpu.sync_copy(msgs, acc.at[dstv], add=True)
            return carry
        lax.fori_loop(0, ITERS, chunk, 0)
        plsc.subcore_barrier()
        for j in range(40):
            r = row0 + j * 16
            pltpu.sync_copy(acc.at[pl.ds(r, 16)], out_hbm.at[core, pl.ds(r, 16)])
    return k


@functools.lru_cache(None)
def _attn_logits(D, nt, K=80):
    TW = D // nt
    GT = TW // 16
    ITERS = EPW // K
    scale = 1.0 / math.sqrt(D)

    scratch = [pltpu.VMEM((K,), jnp.int32), pltpu.VMEM((K,), jnp.int32),
               pltpu.VMEM((K * 4 + 16,), jnp.float32)]
    scratch += [pltpu.VMEM((K, TW), jnp.float32) for _ in range(2 * nt)]
    scratch += [pltpu.VMEM((K,), jnp.float32),
                pltpu.VMEM((4, D), jnp.float32),
                pltpu.VMEM((1, D), jnp.float32),
                pltpu.VMEM((16,), jnp.float32),
                pltpu.SemaphoreType.DMA]

    @functools.partial(
        pl.kernel, mesh=_MESH,
        out_type=(jax.ShapeDtypeStruct((E,), jnp.float32),
                  jax.ShapeDtypeStruct((NW * 16,), jnp.float32)),
        scratch_types=scratch)
    def k(*refs):
        qk_tabs = refs[:2 * nt]
        src_hbm, dst_hbm, attr_hbm, w_hbm, b_hbm = refs[2 * nt:2 * nt + 5]
        logit_hbm, pmax_hbm = refs[2 * nt + 5:2 * nt + 7]
        srcv, dstv, attrv = refs[2 * nt + 7:2 * nt + 10]
        qrows = refs[2 * nt + 10:3 * nt + 10]
        krows = refs[3 * nt + 10:4 * nt + 10]
        lbuf, wv, bv, mbuf, sem = refs[4 * nt + 10:]

        core = lax.axis_index("c")
        sid = lax.axis_index("s")
        wid = sid * NC + core
        pltpu.sync_copy(w_hbm, wv)
        pltpu.sync_copy(b_hbm, bv)
        lane = lax.iota(jnp.int32, 16)
        base0 = wid * EPW

        def chunk(it, mcar):
            base = base0 + it * K
            pltpu.sync_copy(src_hbm.at[pl.ds(base, K)], srcv)
            pltpu.sync_copy(dst_hbm.at[pl.ds(base, K)], dstv)
            pltpu.sync_copy(attr_hbm.at[pl.ds(base * 4, K * 4)], attrv.at[pl.ds(0, K * 4)])
            for t in range(nt):
                pltpu.async_copy(qk_tabs[t].at[dstv], qrows[t], sem).wait()
                pltpu.async_copy(qk_tabs[nt + t].at[srcv], krows[t], sem).wait()

            for j in range(K // 16):
                def edge(ee, lvec, j=j):
                    e = j * 16 + ee
                    a0 = _splat(attrv, e * 4)
                    a1 = _splat(attrv, e * 4 + 1)
                    a2 = _splat(attrv, e * 4 + 2)
                    a3 = _splat(attrv, e * 4 + 3)
                    vacc = jnp.zeros((16,), jnp.float32)
                    for t in range(nt):
                        for g in range(GT):
                            sf = pl.ds(t * TW + g * 16, 16)
                            sl = pl.ds(g * 16, 16)
                            ev = bv[0, sf] + a0 * wv[0, sf] + a1 * wv[1, sf]
                            ev = ev + a2 * wv[2, sf] + a3 * wv[3, sf]
                            vacc = vacc + qrows[t][e, sl] * (krows[t][e, sl] + ev)
                    tval = _hsum16(vacc) * scale
                    return jnp.where(lane == ee, tval, lvec)
                lvec = lax.fori_loop(0, 16, edge,
                                     jnp.zeros((16,), jnp.float32))
                lbuf[pl.ds(j * 16, 16)] = lvec
                mcar = jnp.maximum(mcar, lvec)
            pltpu.sync_copy(lbuf, logit_hbm.at[pl.ds(base, K)])
            return mcar

        m = lax.fori_loop(0, ITERS, chunk,
                          jnp.full((16,), -jnp.inf, jnp.float32))
        mbuf[...] = m
        pltpu.sync_copy(mbuf, pmax_hbm.at[pl.ds(wid * 16, 16)])
    return k


@functools.lru_cache(None)
def _attn_den(K=80):
    ITERS = EPW // K

    @functools.partial(
        pl.kernel, mesh=_MESH,
        out_type=(jax.ShapeDtypeStruct((E,), jnp.float32),
                  jax.ShapeDtypeStruct((NC, N, 128), jnp.float32)),
        scratch_types=[
            pltpu.VMEM((K,), jnp.int32),
            pltpu.VMEM((K,), jnp.float32),
            pltpu.VMEM((K + 16,), jnp.float32),
            pltpu.VMEM((K, 128), jnp.float32),
            pltpu.VMEM((1, 128), jnp.float32),
            pltpu.VMEM((16, 128), jnp.float32),
            pltpu.VMEM_SHARED((N, 128), jnp.float32),
            pltpu.SemaphoreType.DMA,
        ])
    def k(logit_hbm, gmax_hbm, dst_hbm, ex_hbm, den_hbm,
          dstv, lb, exb, msg, gv, zbuf, acc, sem):
        core = lax.axis_index("c")
        sid = lax.axis_index("s")
        wid = sid * NC + core
        pltpu.sync_copy(gmax_hbm, gv)
        gvec = gv[0, pl.ds(0, 16)]
        _zero_fill(zbuf, 16, 8)
        _zero_fill(msg, K, 8)
        row0 = sid * 624
        for j in range(40):
            pltpu.sync_copy(zbuf, acc.at[pl.ds(row0 + j * 16, 16)])
        plsc.subcore_barrier()
        base0 = wid * EPW

        def chunk(it, carry):
            base = base0 + it * K
            pltpu.sync_copy(dst_hbm.at[pl.ds(base, K)], dstv)
            pltpu.sync_copy(logit_hbm.at[pl.ds(base, K)], lb)
            for j in range(K // 16):
                s = pl.ds(j * 16, 16)
                exb[s] = jnp.exp(lb[s] - gvec)

            def edge(e, c2):
                msg[e, pl.ds(0, 16)] = _splat(exb, e)
                return c2
            lax.fori_loop(0, K, edge, 0)
            pltpu.sync_copy(exb.at[pl.ds(0, K)], ex_hbm.at[pl.ds(base, K)])
            pltpu.sync_copy(msg, acc.at[dstv], add=True)
            return carry
        lax.fori_loop(0, ITERS, chunk, 0)
        plsc.subcore_barrier()
        for j in range(40):
            r = row0 + j * 16
            pltpu.sync_copy(acc.at[pl.ds(r, 16)], den_hbm.at[core, pl.ds(r, 16)])
    return k


@functools.lru_cache(None)
def _attn_out(D=128, K=80):
    G = D // 16
    ITERS = EPW // K

    @functools.partial(
        pl.kernel, mesh=_MESH,
        out_type=jax.ShapeDtypeStruct((NC, N, D), jnp.float32),
        scratch_types=[
            pltpu.VMEM((K,), jnp.int32),
            pltpu.VMEM((K,), jnp.int32),
            pltpu.VMEM((K * 4 + 16,), jnp.float32),
            pltpu.VMEM((K, D), jnp.float32),
            pltpu.VMEM((K, 128), jnp.float32),
            pltpu.VMEM((K, 128), jnp.float32),
            pltpu.VMEM((K + 16,), jnp.float32),
            pltpu.VMEM((K, D), jnp.float32),
            pltpu.VMEM((4, D), jnp.float32),
            pltpu.VMEM((1, D), jnp.float32),
            pltpu.VMEM((16, D), jnp.float32),
            pltpu.VMEM_SHARED((N, D), jnp.float32),
            pltpu.SemaphoreType.DMA,
        ])
    def k(vtab, ex_hbm, den0, den1, src_hbm, dst_hbm, attr_hbm, w_hbm, b_hbm,
          out_hbm, srcv, dstv, attrv, vrows, d0, d1, exb, msgs, wv, bv, zbuf,
          acc, sem):
        core = lax.axis_index("c")
        sid = lax.axis_index("s")
        wid = sid * NC + core
        pltpu.sync_copy(w_hbm, wv)
        pltpu.sync_copy(b_hbm, bv)
        _zero_fill(zbuf, 16, G)
        row0 = sid * 624
        for j in range(40):
            pltpu.sync_copy(zbuf, acc.at[pl.ds(row0 + j * 16, 16)])
        plsc.subcore_barrier()
        base0 = wid * EPW

        def chunk(it, carry):
            base = base0 + it * K
            pltpu.sync_copy(src_hbm.at[pl.ds(base, K)], srcv)
            pltpu.sync_copy(dst_hbm.at[pl.ds(base, K)], dstv)
            pltpu.sync_copy(attr_hbm.at[pl.ds(base * 4, K * 4)], attrv.at[pl.ds(0, K * 4)])
            pltpu.sync_copy(ex_hbm.at[pl.ds(base, K)], exb.at[pl.ds(0, K)])
            pltpu.async_copy(vtab.at[srcv], vrows, sem).wait()
            pltpu.async_copy(den0.at[dstv], d0, sem).wait()
            pltpu.async_copy(den1.at[dstv], d1, sem).wait()

            def edge(e, c2):
                a0 = _splat(attrv, e * 4)
                a1 = _splat(attrv, e * 4 + 1)
                a2 = _splat(attrv, e * 4 + 2)
                a3 = _splat(attrv, e * 4 + 3)
                s16 = pl.ds(0, 16)
                den = d0[e, s16] + d1[e, s16] + 1e-16
                alpha = _splat(exb, e) / den
                for g in range(G):
                    s = pl.ds(g * 16, 16)
                    ev = bv[0, s] + a0 * wv[0, s] + a1 * wv[1, s]
                    ev = ev + a2 * wv[2, s] + a3 * wv[3, s]
                    msgs[e, s] = alpha * (vrows[e, s] + ev)
                return c2
            lax.fori_loop(0, K, edge, 0)
            pltpu.sync_copy(msgs, acc.at[dstv], add=True)
            return carry
        lax.fori_loop(0, ITERS, chunk, 0)
        plsc.subcore_barrier()
        for j in range(40):
            r = row0 + j * 16
            pltpu.sync_copy(acc.at[pl.ds(r, 16)], out_hbm.at[core, pl.ds(r, 16)])
    return k


@functools.lru_cache(None)
def _edge_relu(D=1024, K=16):
    G = D // 16
    ITERS = EPW // K

    @functools.partial(
        pl.kernel, mesh=_MESH,
        out_type=jax.ShapeDtypeStruct((E, D), jnp.float32),
        scratch_types=[
            pltpu.VMEM((K,), jnp.int32),
            pltpu.VMEM((K,), jnp.int32),
            pltpu.VMEM((K, D), jnp.float32),
            pltpu.VMEM((K, D), jnp.float32),
            pltpu.VMEM((K, D), jnp.float32),
            pltpu.SemaphoreType.DMA,
        ])
    def k(atab, btab, src_hbm, dst_hbm, z_hbm, srcv, dstv, ar, br, zb, sem):
        core = lax.axis_index("c")
        sid = lax.axis_index("s")
        wid = sid * NC + core
        base0 = wid * EPW

        def chunk(it, carry):
            base = base0 + it * K
            pltpu.sync_copy(src_hbm.at[pl.ds(base, K)], srcv)
            pltpu.sync_copy(dst_hbm.at[pl.ds(base, K)], dstv)
            pltpu.async_copy(atab.at[srcv], ar, sem).wait()
            pltpu.async_copy(btab.at[dstv], br, sem).wait()

            def edge(e, c2):
                for g in range(G):
                    s = pl.ds(g * 16, 16)
                    zb[e, s] = jnp.maximum(ar[e, s] + br[e, s], 0.0)
                return c2
            lax.fori_loop(0, K, edge, 0)
            pltpu.sync_copy(zb, z_hbm.at[pl.ds(base, K), :])
            return carry
        lax.fori_loop(0, ITERS, chunk, 0)
    return k


def _act(name, y):
    if name == "relu":
        return jnp.maximum(y, 0.0)
    if name == "selu":
        return _SELU_L * jnp.where(y > 0, y, _SELU_A * (jnp.exp(y) - 1.0))
    if name == "softmax":
        m = jnp.max(y, axis=-1, keepdims=True)
        ex = jnp.exp(y - m)
        return ex / jnp.sum(ex, axis=-1, keepdims=True)
    return y


def _tc_apply(chunks, adds, branches, rb):
    M = chunks[0].shape[0]
    grid = M // rb
    assert grid * rb == M
    inputs, in_specs = [], []

    def add_in(arr, blk_rows):
        inputs.append(arr)
        in_specs.append(pl.BlockSpec((blk_rows, arr.shape[1]),
                                     lambda i: (i, 0)))

    def add_const(arr):
        inputs.append(arr)
        in_specs.append(pl.BlockSpec(arr.shape, lambda i: (0, 0)))

    meta_chunks = []
    for ci, c in enumerate(chunks):
        add_in(c, rb)
        for a in adds[ci]:
            add_in(a, rb)
        meta_chunks.append(1 + len(adds[ci]))
    meta_br = []
    out_shapes, out_specs = [], []
    for br in branches:
        for (w, b) in br["layers"]:
            add_const(w)
            add_const(b)
        for pa in br.get("post", []):
            add_in(pa, rb)
        dlast = br["layers"][-1][0].shape[1]
        ow = br.get("out_widths") or [dlast]
        for w_ in ow:
            out_shapes.append(
                jax.ShapeDtypeStruct((M, w_), jnp.float32))
            out_specs.append(pl.BlockSpec((rb, w_), lambda i: (i, 0)))
        meta_br.append((len(br["layers"]), len(br.get("post", [])), len(ow)))

    def body(*refs):
        pos = 0
        xs = []
        for cnt in meta_chunks:
            xc = refs[pos][...]
            for j in range(1, cnt):
                xc = xc + refs[pos + j][...]
            xs.append(xc)
            pos += cnt
        x = xs[0] if len(xs) == 1 else jnp.concatenate(xs, axis=-1)
        opos = len(inputs)
        for bi, br in enumerate(branches):
            nl, npost, nout = meta_br[bi]
            y = x
            for li in range(nl):
                w_ref = refs[pos]
                b_ref = refs[pos + 1]
                pos += 2
                y = jnp.dot(y, w_ref[...],
                            preferred_element_type=jnp.float32) + b_ref[...]
                if li == nl - 1:
                    for j in range(npost):
                        y = y + refs[pos + j][...]
                y = _act(br["acts"][li], y)
            pos += npost
            off = 0
            ow = br.get("out_widths") or [y.shape[-1]]
            for w_ in ow:
                refs[opos][...] = y[:, off:off + w_]
                off += w_
                opos += 1

    outs = pl.pallas_call(
        body, grid=(grid,), in_specs=in_specs, out_specs=out_specs,
        out_shape=out_shapes)(*inputs)
    return list(outs) if isinstance(outs, (list, tuple)) else [outs]


def _tc_gmax(vals):
    rows = vals.shape[0] // 128

    def body(x_ref, o_ref):
        o_ref[...] = jnp.broadcast_to(jnp.max(x_ref[...]), (1, 128))
    return pl.pallas_call(
        body, grid=(1,),
        in_specs=[pl.BlockSpec((rows, 128), lambda i: (0, 0))],
        out_specs=pl.BlockSpec((1, 128), lambda i: (0, 0)),
        out_shape=jax.ShapeDtypeStruct((1, 128), jnp.float32),
    )(vals.reshape(rows, 128))


def _b2(p):
    return p["b"].reshape(1, -1)


def _gine_layer(h_chunks, src, dst, attr, p, acts):
    parts = []
    off = 0
    wle = p["le"]["w"]
    ble = p["le"]["b"]
    for hc in h_chunks:
        w = hc.shape[1]
        pr = _gine_agg(w)(hc, src, dst, attr,
                          wle[:, off:off + w],
                          ble[off:off + w].reshape(1, w))
        parts.append([pr[0], pr[1]])
        off += w
    layers = [(l["w"], _b2(l)) for l in p["nn"]]
    dlast = layers[-1][0].shape[1]
    ow = [128] * (dlast // 128) if dlast > 128 else None
    out = _tc_apply(h_chunks, parts,
                    [dict(layers=layers, acts=acts, post=[],
                          out_widths=ow)], 400)
    return out


def _tconv_layer(h_chunks, src, dst, attr, p):
    D = sum(c.shape[1] for c in h_chunks)
    nt = D // 128
    wq, wk, wv_ = p["q"]["w"], p["k"]["w"], p["v"]["w"]
    wqkv = jnp.concatenate([wq, wk, wv_], axis=1)
    bqkv = jnp.concatenate([p["q"]["b"], p["k"]["b"], p["v"]["b"]])
    qkv = _tc_apply(h_chunks, [[] for _ in h_chunks],
                    [dict(layers=[(wqkv, bqkv.reshape(1, -1))], acts=["none"],
                          post=[], out_widths=[128] * (3 * nt))], 400)
    qtabs = qkv[:nt]
    ktabs = qkv[nt:2 * nt]
    vtabs = qkv[2 * nt:]

    we, be = p["e"]["w"], p["e"]["b"]
    logits, pmax = _attn_logits(D, nt)(
        *qtabs, *ktabs, src, dst, attr, we, be.reshape(1, D))
    gmaxb = _tc_gmax(pmax)
    ex, den = _attn_den()(logits, gmaxb, dst)
    parts = []
    for t in range(nt):
        pr = _attn_out()(vtabs[t], ex, den[0], den[1], src, dst, attr,
                         we[:, t * 128:(t + 1) * 128],
                         be[t * 128:(t + 1) * 128].reshape(1, 128))
        parts.append(pr)
    ws, bs = p["s"]["w"], p["s"]["b"]
    branches = []
    for t in range(nt):
        branches.append(dict(layers=[(ws[:, t * 128:(t + 1) * 128],
                                      bs[t * 128:(t + 1) * 128].reshape(1, 128))],
                             acts=["selu"],
                             post=[parts[t][0], parts[t][1]],
                             out_widths=None))
    return _tc_apply(h_chunks, [[] for _ in h_chunks], branches, 400)


def _pad_cols(l, n):
    w, b = l["w"], l["b"]
    return {"w": jnp.pad(w, ((0, 0), (0, n - w.shape[1]))),
            "b": jnp.pad(b, (0, n - b.shape[0]))}


def _pad_rows(l, n):
    w = l["w"]
    return {"w": jnp.pad(w, ((0, n - w.shape[0]), (0, 0))), "b": l["b"]}


def kernel(x, edge_index, edge_attr, params):
    src = edge_index[0]
    dst = edge_index[1]
    attr = edge_attr.reshape(-1)
    p = dict(params)
    p["c3"] = {"le": p["c3"]["le"],
               "nn": [_pad_cols(p["c3"]["nn"][0], 128)]}
    p["c4"] = {"le": _pad_cols(p["c4"]["le"], 128),
               "nn": [_pad_rows(p["c4"]["nn"][0], 128)]}
    p["c5"] = {"le": p["c5"]["le"],
               "nn": [_pad_cols(p["c5"]["nn"][0], 128)]}
    p["c6"] = {"le": _pad_cols(p["c6"]["le"], 128),
               "nn": [_pad_rows(p["c6"]["nn"][0], 128)]}

    h = _gine_layer([x], src, dst, attr, p["c1"], ["relu", "selu"])
    h = _gine_layer(h, src, dst, attr, p["c2"], ["relu", "selu"])
    h = _gine_layer(h, src, dst, attr, p["c3"], ["selu"])
    h = _gine_layer(h, src, dst, attr, p["c4"], ["selu"])
    h = _tconv_layer(h, src, dst, attr, p["t1"])
    h = _gine_layer(h, src, dst, attr, p["c5"], ["selu"])
    h = _gine_layer(h, src, dst, attr, p["c6"], ["selu"])
    h = _gine_layer(h, src, dst, attr, p["c7"], ["selu"])
    h = _gine_layer(h, src, dst, attr, p["c8"], ["selu"])
    h = _tconv_layer(h, src, dst, attr, p["t2"])
    h = _gine_layer(h, src, dst, attr, p["c9"], ["selu"])

    mlp = p["mlp"]
    w1, b1 = mlp[0]["w"], mlp[0]["b"]
    wcat = jnp.concatenate([w1[:512], w1[512:]], axis=1)
    bcat = jnp.concatenate([b1, jnp.zeros_like(b1)]).reshape(1, 2048)
    ab = _tc_apply(h, [[] for _ in h],
                   [dict(layers=[(wcat, bcat)], acts=["none"],
                         out_widths=[1024, 1024])], 400)
    z1 = _edge_relu()(ab[0], ab[1], src, dst)

    w3 = mlp[2]["w"]
    b3 = mlp[2]["b"]
    w3p = jnp.zeros((128, 128), jnp.float32).at[:, :2].set(w3)
    b3p = jnp.full((1, 128), -1e30, jnp.float32).at[0, :2].set(b3)
    z3 = _tc_apply([z1], [[]],
                   [dict(layers=[(mlp[1]["w"], _b2(mlp[1])), (w3p, b3p)],
                         acts=["selu", "softmax"], post=[],
                         out_widths=None)], 512)[0]
    return z3[:, :2]


# speedup vs baseline: 1.4907x; 1.1303x over previous
import functools
import math

import jax
import jax.numpy as jnp
from jax import lax
from jax.experimental import pallas as pl
from jax.experimental.pallas import tpu as pltpu
from jax.experimental.pallas import tpu_sc as plsc

N = 10000
E = 320000
NC = 2
NS = 16
NW = NC * NS
EPW = E // NW
NROW = N // NS
ZR = 125

_MESH = plsc.VectorSubcoreMesh(
    core_axis_name="c", subcore_axis_name="s", num_cores=NC, num_subcores=NS)

_SELU_L = 1.0507009873554804934193349852946
_SELU_A = 1.6732632423543772848170429916717


def _hsum16(v):
    t = [v[i] for i in range(16)]
    while len(t) > 1:
        t = [t[i] + t[i + 1] for i in range(0, len(t), 2)]
    return t[0]


def _splat(ref1d, i):
    v = ref1d[pl.ds(i, 16)]
    return jnp.full((16,), v[0], jnp.float32)


def _zero_fill(buf, rows, groups):
    def body(r, carry):
        for g in range(groups):
            buf[r, pl.ds(g * 16, 16)] = jnp.zeros((16,), jnp.float32)
        return carry
    lax.fori_loop(0, rows, body, 0)


NBLK = E // 640
BPW = (NBLK + NW - 1) // NW


@functools.lru_cache(None)
def _gine_agg(D):
    G = D // 16

    @functools.partial(
        pl.kernel, mesh=_MESH,
        out_type=jax.ShapeDtypeStruct((NC, N, D), jnp.float32),
        scratch_types=[
            pltpu.VMEM((640,), jnp.int32),
            pltpu.VMEM((640,), jnp.int32),
            pltpu.VMEM((2560 + 16,), jnp.float32),
            pltpu.VMEM((3, 80, D), jnp.float32),
            pltpu.VMEM((4, D), jnp.float32),
            pltpu.VMEM((1, D), jnp.float32),
            pltpu.VMEM((16, D), jnp.float32),
            pltpu.VMEM_SHARED((N, D), jnp.float32),
            pltpu.SemaphoreType.DMA,
            pltpu.SemaphoreType.DMA,
        ])
    def k(h_hbm, src_hbm, dst_hbm, attr_hbm, w_hbm, b_hbm, out_hbm,
          srcv, dstv, attrv, grows, wv, bv, zbuf, acc, semg, sems):
        core = lax.axis_index("c")
        sid = lax.axis_index("s")
        wid = sid * NC + core
        pltpu.sync_copy(w_hbm, wv)
        pltpu.sync_copy(b_hbm, bv)
        _zero_fill(zbuf, 16, G)
        row0 = sid * 624
        for j in range(40):
            pltpu.sync_copy(zbuf, acc.at[pl.ds(row0 + j * 16, 16)])
        plsc.subcore_barrier()

        def sblk(t, carry):
            blk = t * NW + wid

            @pl.when(blk < NBLK)
            def _():
                pltpu.sync_copy(src_hbm.at[pl.ds(blk * 640, 640)], srcv)
                pltpu.sync_copy(dst_hbm.at[pl.ds(blk * 640, 640)], dstv)
                pltpu.sync_copy(attr_hbm.at[pl.ds(blk * 2560, 2560)],
                                attrv.at[pl.ds(0, 2560)])
                def fire_g(j):
                    return pltpu.async_copy(
                        h_hbm.at[srcv.at[pl.ds(j * 80, 80)]],
                        grows.at[j % 3], semg)

                def fire_s(j):
                    return pltpu.async_copy(
                        grows.at[j % 3],
                        acc.at[dstv.at[pl.ds(j * 80, 80)]], sems, add=True)

                def comp(j):
                    buf = grows.at[j % 3]

                    def edge(e, c2):
                        ea = j * 80 + e
                        a0 = _splat(attrv, ea * 4)
                        a1 = _splat(attrv, ea * 4 + 1)
                        a2 = _splat(attrv, ea * 4 + 2)
                        a3 = _splat(attrv, ea * 4 + 3)
                        for g in range(G):
                            sg = pl.ds(g * 16, 16)
                            m = buf[e, sg] + bv[0, sg]
                            m = m + a0 * wv[0, sg] + a1 * wv[1, sg]
                            m = m + a2 * wv[2, sg] + a3 * wv[3, sg]
                            buf[e, sg] = jnp.maximum(m, 0.0)
                        return c2
                    lax.fori_loop(0, 80, edge, 0)

                cg = {0: fire_g(0)}
                cs = {}
                for j in range(8):
                    if j + 1 < 8:
                        if j - 2 >= 0:
                            cs[j - 2].wait()
                        cg[j + 1] = fire_g(j + 1)
                    cg[j].wait()
                    comp(j)
                    cs[j] = fire_s(j)
                for j in (5, 6, 7):
                    cs[j].wait()
            return carry
        lax.fori_loop(0, BPW, sblk, 0)
        plsc.subcore_barrier()
        for j in range(40):
            r = row0 + j * 16
            pltpu.sync_copy(acc.at[pl.ds(r, 16)], out_hbm.at[core, pl.ds(r, 16)])
    return k


@functools.lru_cache(None)
def _attn_logits(D, nt, K=80):
    TW = D // nt
    GT = TW // 16
    ITERS = EPW // K
    scale = 1.0 / math.sqrt(D)

    scratch = [pltpu.VMEM((K,), jnp.int32), pltpu.VMEM((K,), jnp.int32),
               pltpu.VMEM((K * 4 + 16,), jnp.float32)]
    scratch += [pltpu.VMEM((K, TW), jnp.float32) for _ in range(2 * nt)]
    scratch += [pltpu.VMEM((K,), jnp.float32),
                pltpu.VMEM((4, D), jnp.float32),
                pltpu.VMEM((1, D), jnp.float32),
                pltpu.VMEM((16,), jnp.float32),
                pltpu.SemaphoreType.DMA]

    @functools.partial(
        pl.kernel, mesh=_MESH,
        out_type=(jax.ShapeDtypeStruct((E,), jnp.float32),
                  jax.ShapeDtypeStruct((NW * 16,), jnp.float32)),
        scratch_types=scratch)
    def k(*refs):
        qk_tabs = refs[:2 * nt]
        src_hbm, dst_hbm, attr_hbm, w_hbm, b_hbm = refs[2 * nt:2 * nt + 5]
        logit_hbm, pmax_hbm = refs[2 * nt + 5:2 * nt + 7]
        srcv, dstv, attrv = refs[2 * nt + 7:2 * nt + 10]
        qrows = refs[2 * nt + 10:3 * nt + 10]
        krows = refs[3 * nt + 10:4 * nt + 10]
        lbuf, wv, bv, mbuf, sem = refs[4 * nt + 10:]

        core = lax.axis_index("c")
        sid = lax.axis_index("s")
        wid = sid * NC + core
        pltpu.sync_copy(w_hbm, wv)
        pltpu.sync_copy(b_hbm, bv)
        lane = lax.iota(jnp.int32, 16)
        base0 = wid * EPW

        def chunk(it, mcar):
            base = base0 + it * K
            pltpu.sync_copy(src_hbm.at[pl.ds(base, K)], srcv)
            pltpu.sync_copy(dst_hbm.at[pl.ds(base, K)], dstv)
            pltpu.sync_copy(attr_hbm.at[pl.ds(base * 4, K * 4)], attrv.at[pl.ds(0, K * 4)])
            for t in range(nt):
                pltpu.async_copy(qk_tabs[t].at[dstv], qrows[t], sem).wait()
                pltpu.async_copy(qk_tabs[nt + t].at[srcv], krows[t], sem).wait()

            for j in range(K // 16):
                def edge(ee, lvec, j=j):
                    e = j * 16 + ee
                    a0 = _splat(attrv, e * 4)
                    a1 = _splat(attrv, e * 4 + 1)
                    a2 = _splat(attrv, e * 4 + 2)
                    a3 = _splat(attrv, e * 4 + 3)
                    vacc = jnp.zeros((16,), jnp.float32)
                    for t in range(nt):
                        for g in range(GT):
                            sf = pl.ds(t * TW + g * 16, 16)
                            sl = pl.ds(g * 16, 16)
                            ev = bv[0, sf] + a0 * wv[0, sf] + a1 * wv[1, sf]
                            ev = ev + a2 * wv[2, sf] + a3 * wv[3, sf]
                            vacc = vacc + qrows[t][e, sl] * (krows[t][e, sl] + ev)
                    tval = _hsum16(vacc) * scale
                    return jnp.where(lane == ee, tval, lvec)
                lvec = lax.fori_loop(0, 16, edge,
                                     jnp.zeros((16,), jnp.float32))
                lbuf[pl.ds(j * 16, 16)] = lvec
                mcar = jnp.maximum(mcar, lvec)
            pltpu.sync_copy(lbuf, logit_hbm.at[pl.ds(base, K)])
            return mcar

        m = lax.fori_loop(0, ITERS, chunk,
                          jnp.full((16,), -jnp.inf, jnp.float32))
        mbuf[...] = m
        pltpu.sync_copy(mbuf, pmax_hbm.at[pl.ds(wid * 16, 16)])
    return k


@functools.lru_cache(None)
def _attn_den(K=80):
    ITERS = EPW // K

    @functools.partial(
        pl.kernel, mesh=_MESH,
        out_type=(jax.ShapeDtypeStruct((E,), jnp.float32),
                  jax.ShapeDtypeStruct((NC, N, 128), jnp.float32)),
        scratch_types=[
            pltpu.VMEM((K,), jnp.int32),
            pltpu.VMEM((K,), jnp.float32),
            pltpu.VMEM((K + 16,), jnp.float32),
            pltpu.VMEM((K, 128), jnp.float32),
            pltpu.VMEM((1, 128), jnp.float32),
            pltpu.VMEM((16, 128), jnp.float32),
            pltpu.VMEM_SHARED((N, 128), jnp.float32),
            pltpu.SemaphoreType.DMA,
        ])
    def k(logit_hbm, gmax_hbm, dst_hbm, ex_hbm, den_hbm,
          dstv, lb, exb, msg, gv, zbuf, acc, sem):
        core = lax.axis_index("c")
        sid = lax.axis_index("s")
        wid = sid * NC + core
        pltpu.sync_copy(gmax_hbm, gv)
        gvec = gv[0, pl.ds(0, 16)]
        _zero_fill(zbuf, 16, 8)
        _zero_fill(msg, K, 8)
        row0 = sid * 624
        for j in range(40):
            pltpu.sync_copy(zbuf, acc.at[pl.ds(row0 + j * 16, 16)])
        plsc.subcore_barrier()
        base0 = wid * EPW

        def chunk(it, carry):
            base = base0 + it * K
            pltpu.sync_copy(dst_hbm.at[pl.ds(base, K)], dstv)
            pltpu.sync_copy(logit_hbm.at[pl.ds(base, K)], lb)
            for j in range(K // 16):
                s = pl.ds(j * 16, 16)
                exb[s] = jnp.exp(lb[s] - gvec)

            def edge(e, c2):
                msg[e, pl.ds(0, 16)] = _splat(exb, e)
                return c2
            lax.fori_loop(0, K, edge, 0)
            pltpu.sync_copy(exb.at[pl.ds(0, K)], ex_hbm.at[pl.ds(base, K)])
            pltpu.sync_copy(msg, acc.at[dstv], add=True)
            return carry
        lax.fori_loop(0, ITERS, chunk, 0)
        plsc.subcore_barrier()
        for j in range(40):
            r = row0 + j * 16
            pltpu.sync_copy(acc.at[pl.ds(r, 16)], den_hbm.at[core, pl.ds(r, 16)])
    return k


@functools.lru_cache(None)
def _attn_out(D=128, K=80):
    G = D // 16
    ITERS = EPW // K

    @functools.partial(
        pl.kernel, mesh=_MESH,
        out_type=jax.ShapeDtypeStruct((NC, N, D), jnp.float32),
        scratch_types=[
            pltpu.VMEM((K,), jnp.int32),
            pltpu.VMEM((K,), jnp.int32),
            pltpu.VMEM((K * 4 + 16,), jnp.float32),
            pltpu.VMEM((K, D), jnp.float32),
            pltpu.VMEM((K, 128), jnp.float32),
            pltpu.VMEM((K, 128), jnp.float32),
            pltpu.VMEM((K + 16,), jnp.float32),
            pltpu.VMEM((K, D), jnp.float32),
            pltpu.VMEM((4, D), jnp.float32),
            pltpu.VMEM((1, D), jnp.float32),
            pltpu.VMEM((16, D), jnp.float32),
            pltpu.VMEM_SHARED((N, D), jnp.float32),
            pltpu.SemaphoreType.DMA,
        ])
    def k(vtab, ex_hbm, den0, den1, src_hbm, dst_hbm, attr_hbm, w_hbm, b_hbm,
          out_hbm, srcv, dstv, attrv, vrows, d0, d1, exb, msgs, wv, bv, zbuf,
          acc, sem):
        core = lax.axis_index("c")
        sid = lax.axis_index("s")
        wid = sid * NC + core
        pltpu.sync_copy(w_hbm, wv)
        pltpu.sync_copy(b_hbm, bv)
        _zero_fill(zbuf, 16, G)
        row0 = sid * 624
        for j in range(40):
            pltpu.sync_copy(zbuf, acc.at[pl.ds(row0 + j * 16, 16)])
        plsc.subcore_barrier()
        base0 = wid * EPW

        def chunk(it, carry):
            base = base0 + it * K
            pltpu.sync_copy(src_hbm.at[pl.ds(base, K)], srcv)
            pltpu.sync_copy(dst_hbm.at[pl.ds(base, K)], dstv)
            pltpu.sync_copy(attr_hbm.at[pl.ds(base * 4, K * 4)], attrv.at[pl.ds(0, K * 4)])
            pltpu.sync_copy(ex_hbm.at[pl.ds(base, K)], exb.at[pl.ds(0, K)])
            pltpu.async_copy(vtab.at[srcv], vrows, sem).wait()
            pltpu.async_copy(den0.at[dstv], d0, sem).wait()
            pltpu.async_copy(den1.at[dstv], d1, sem).wait()

            def edge(e, c2):
                a0 = _splat(attrv, e * 4)
                a1 = _splat(attrv, e * 4 + 1)
                a2 = _splat(attrv, e * 4 + 2)
                a3 = _splat(attrv, e * 4 + 3)
                s16 = pl.ds(0, 16)
                den = d0[e, s16] + d1[e, s16] + 1e-16
                alpha = _splat(exb, e) / den
                for g in range(G):
                    s = pl.ds(g * 16, 16)
                    ev = bv[0, s] + a0 * wv[0, s] + a1 * wv[1, s]
                    ev = ev + a2 * wv[2, s] + a3 * wv[3, s]
                    msgs[e, s] = alpha * (vrows[e, s] + ev)
                return c2
            lax.fori_loop(0, K, edge, 0)
            pltpu.sync_copy(msgs, acc.at[dstv], add=True)
            return carry
        lax.fori_loop(0, ITERS, chunk, 0)
        plsc.subcore_barrier()
        for j in range(40):
            r = row0 + j * 16
            pltpu.sync_copy(acc.at[pl.ds(r, 16)], out_hbm.at[core, pl.ds(r, 16)])
    return k


@functools.lru_cache(None)
def _edge_relu(D=1024, K=16):
    G = D // 16
    ITERS = EPW // K

    @functools.partial(
        pl.kernel, mesh=_MESH,
        out_type=jax.ShapeDtypeStruct((E, D), jnp.float32),
        scratch_types=[
            pltpu.VMEM((K,), jnp.int32),
            pltpu.VMEM((K,), jnp.int32),
            pltpu.VMEM((K, D), jnp.float32),
            pltpu.VMEM((K, D), jnp.float32),
            pltpu.VMEM((K, D), jnp.float32),
            pltpu.SemaphoreType.DMA,
        ])
    def k(atab, btab, src_hbm, dst_hbm, z_hbm, srcv, dstv, ar, br, zb, sem):
        core = lax.axis_index("c")
        sid = lax.axis_index("s")
        wid = sid * NC + core
        base0 = wid * EPW

        def chunk(it, carry):
            base = base0 + it * K
            pltpu.sync_copy(src_hbm.at[pl.ds(base, K)], srcv)
            pltpu.sync_copy(dst_hbm.at[pl.ds(base, K)], dstv)
            pltpu.async_copy(atab.at[srcv], ar, sem).wait()
            pltpu.async_copy(btab.at[dstv], br, sem).wait()

            def edge(e, c2):
                for g in range(G):
                    s = pl.ds(g * 16, 16)
                    zb[e, s] = jnp.maximum(ar[e, s] + br[e, s], 0.0)
                return c2
            lax.fori_loop(0, K, edge, 0)
            pltpu.sync_copy(zb, z_hbm.at[pl.ds(base, K), :])
            return carry
        lax.fori_loop(0, ITERS, chunk, 0)
    return k


def _act(name, y):
    if name == "relu":
        return jnp.maximum(y, 0.0)
    if name == "selu":
        return _SELU_L * jnp.where(y > 0, y, _SELU_A * (jnp.exp(y) - 1.0))
    if name == "softmax":
        m = jnp.max(y, axis=-1, keepdims=True)
        ex = jnp.exp(y - m)
        return ex / jnp.sum(ex, axis=-1, keepdims=True)
    return y


def _tc_apply(chunks, adds, branches, rb):
    M = chunks[0].shape[0]
    grid = M // rb
    assert grid * rb == M
    inputs, in_specs = [], []

    def add_in(arr, blk_rows):
        inputs.append(arr)
        in_specs.append(pl.BlockSpec((blk_rows, arr.shape[1]),
                                     lambda i: (i, 0)))

    def add_const(arr):
        inputs.append(arr)
        in_specs.append(pl.BlockSpec(arr.shape, lambda i: (0, 0)))

    meta_chunks = []
    for ci, c in enumerate(chunks):
        add_in(c, rb)
        for a in adds[ci]:
            add_in(a, rb)
        meta_chunks.append(1 + len(adds[ci]))
    meta_br = []
    out_shapes, out_specs = [], []
    for br in branches:
        for (w, b) in br["layers"]:
            add_const(w)
            add_const(b)
        for pa in br.get("post", []):
            add_in(pa, rb)
        dlast = br["layers"][-1][0].shape[1]
        ow = br.get("out_widths") or [dlast]
        for w_ in ow:
            out_shapes.append(
                jax.ShapeDtypeStruct((M, w_), jnp.float32))
            out_specs.append(pl.BlockSpec((rb, w_), lambda i: (i, 0)))
        meta_br.append((len(br["layers"]), len(br.get("post", [])), len(ow)))

    def body(*refs):
        pos = 0
        xs = []
        for cnt in meta_chunks:
            xc = refs[pos][...]
            for j in range(1, cnt):
                xc = xc + refs[pos + j][...]
            xs.append(xc)
            pos += cnt
        x = xs[0] if len(xs) == 1 else jnp.concatenate(xs, axis=-1)
        opos = len(inputs)
        for bi, br in enumerate(branches):
            nl, npost, nout = meta_br[bi]
            y = x
            for li in range(nl):
                w_ref = refs[pos]
                b_ref = refs[pos + 1]
                pos += 2
                y = jnp.dot(y, w_ref[...],
                            preferred_element_type=jnp.float32) + b_ref[...]
                if li == nl - 1:
                    for j in range(npost):
                        y = y + refs[pos + j][...]
                y = _act(br["acts"][li], y)
            pos += npost
            off = 0
            ow = br.get("out_widths") or [y.shape[-1]]
            for w_ in ow:
                refs[opos][...] = y[:, off:off + w_]
                off += w_
                opos += 1

    outs = pl.pallas_call(
        body, grid=(grid,), in_specs=in_specs, out_specs=out_specs,
        out_shape=out_shapes)(*inputs)
    return list(outs) if isinstance(outs, (list, tuple)) else [outs]


def _tc_gmax(vals):
    rows = vals.shape[0] // 128

    def body(x_ref, o_ref):
        o_ref[...] = jnp.broadcast_to(jnp.max(x_ref[...]), (1, 128))
    return pl.pallas_call(
        body, grid=(1,),
        in_specs=[pl.BlockSpec((rows, 128), lambda i: (0, 0))],
        out_specs=pl.BlockSpec((1, 128), lambda i: (0, 0)),
        out_shape=jax.ShapeDtypeStruct((1, 128), jnp.float32),
    )(vals.reshape(rows, 128))


def _b2(p):
    return p["b"].reshape(1, -1)


def _gine_layer(h_chunks, src, dst, attr, p, acts):
    parts = []
    off = 0
    wle = p["le"]["w"]
    ble = p["le"]["b"]
    for hc in h_chunks:
        w = hc.shape[1]
        pr = _gine_agg(w)(hc, src, dst, attr, wle[:, off:off + w],
                          ble[off:off + w].reshape(1, w))
        parts.append([pr[0], pr[1]])
        off += w
    layers = [(l["w"], _b2(l)) for l in p["nn"]]
    dlast = layers[-1][0].shape[1]
    ow = [128] * (dlast // 128) if dlast > 128 else None
    out = _tc_apply(h_chunks, parts,
                    [dict(layers=layers, acts=acts, post=[],
                          out_widths=ow)], 400)
    return out


def _tconv_layer(h_chunks, src, dst, attr, p):
    D = sum(c.shape[1] for c in h_chunks)
    nt = D // 128
    wq, wk, wv_ = p["q"]["w"], p["k"]["w"], p["v"]["w"]
    wqkv = jnp.concatenate([wq, wk, wv_], axis=1)
    bqkv = jnp.concatenate([p["q"]["b"], p["k"]["b"], p["v"]["b"]])
    qkv = _tc_apply(h_chunks, [[] for _ in h_chunks],
                    [dict(layers=[(wqkv, bqkv.reshape(1, -1))], acts=["none"],
                          post=[], out_widths=[128] * (3 * nt))], 400)
    qtabs = qkv[:nt]
    ktabs = qkv[nt:2 * nt]
    vtabs = qkv[2 * nt:]

    we, be = p["e"]["w"], p["e"]["b"]
    logits, pmax = _attn_logits(D, nt)(
        *qtabs, *ktabs, src, dst, attr, we, be.reshape(1, D))
    gmaxb = _tc_gmax(pmax)
    ex, den = _attn_den()(logits, gmaxb, dst)
    parts = []
    for t in range(nt):
        pr = _attn_out()(vtabs[t], ex, den[0], den[1], src, dst, attr,
                         we[:, t * 128:(t + 1) * 128],
                         be[t * 128:(t + 1) * 128].reshape(1, 128))
        parts.append(pr)
    ws, bs = p["s"]["w"], p["s"]["b"]
    branches = []
    for t in range(nt):
        branches.append(dict(layers=[(ws[:, t * 128:(t + 1) * 128],
                                      bs[t * 128:(t + 1) * 128].reshape(1, 128))],
                             acts=["selu"],
                             post=[parts[t][0], parts[t][1]],
                             out_widths=None))
    return _tc_apply(h_chunks, [[] for _ in h_chunks], branches, 400)


def _pad_cols(l, n):
    w, b = l["w"], l["b"]
    return {"w": jnp.pad(w, ((0, 0), (0, n - w.shape[1]))),
            "b": jnp.pad(b, (0, n - b.shape[0]))}


def _pad_rows(l, n):
    w = l["w"]
    return {"w": jnp.pad(w, ((0, n - w.shape[0]), (0, 0))), "b": l["b"]}


def kernel(x, edge_index, edge_attr, params):
    src = edge_index[0]
    dst = edge_index[1]
    attr = edge_attr.reshape(-1)
    p = dict(params)
    p["c3"] = {"le": p["c3"]["le"],
               "nn": [_pad_cols(p["c3"]["nn"][0], 128)]}
    p["c4"] = {"le": _pad_cols(p["c4"]["le"], 128),
               "nn": [_pad_rows(p["c4"]["nn"][0], 128)]}
    p["c5"] = {"le": p["c5"]["le"],
               "nn": [_pad_cols(p["c5"]["nn"][0], 128)]}
    p["c6"] = {"le": _pad_cols(p["c6"]["le"], 128),
               "nn": [_pad_rows(p["c6"]["nn"][0], 128)]}

    h = _gine_layer([x], src, dst, attr, p["c1"], ["relu", "selu"])
    h = _gine_layer(h, src, dst, attr, p["c2"], ["relu", "selu"])
    h = _gine_layer(h, src, dst, attr, p["c3"], ["selu"])
    h = _gine_layer(h, src, dst, attr, p["c4"], ["selu"])
    h = _tconv_layer(h, src, dst, attr, p["t1"])
    h = _gine_layer(h, src, dst, attr, p["c5"], ["selu"])
    h = _gine_layer(h, src, dst, attr, p["c6"], ["selu"])
    h = _gine_layer(h, src, dst, attr, p["c7"], ["selu"])
    h = _gine_layer(h, src, dst, attr, p["c8"], ["selu"])
    h = _tconv_layer(h, src, dst, attr, p["t2"])
    h = _gine_layer(h, src, dst, attr, p["c9"], ["selu"])

    mlp = p["mlp"]
    w1, b1 = mlp[0]["w"], mlp[0]["b"]
    wcat = jnp.concatenate([w1[:512], w1[512:]], axis=1)
    bcat = jnp.concatenate([b1, jnp.zeros_like(b1)]).reshape(1, 2048)
    ab = _tc_apply(h, [[] for _ in h],
                   [dict(layers=[(wcat, bcat)], acts=["none"],
                         out_widths=[1024, 1024])], 400)
    z1 = _edge_relu()(ab[0], ab[1], src, dst)

    w3 = mlp[2]["w"]
    b3 = mlp[2]["b"]
    w3p = jnp.zeros((128, 128), jnp.float32).at[:, :2].set(w3)
    b3p = jnp.full((1, 128), -1e30, jnp.float32).at[0, :2].set(b3)
    z3 = _tc_apply([z1], [[]],
                   [dict(layers=[(mlp[1]["w"], _b2(mlp[1])), (w3p, b3p)],
                         acts=["selu", "softmax"], post=[],
                         out_widths=None)], 512)[0]
    return z3[:, :2]
